# baseline (speedup 1.0000x reference)
import functools

import jax
import jax.numpy as jnp
from jax import lax
from jax.experimental import pallas as pl
from jax.experimental.pallas import tpu as pltpu
from jax.experimental.pallas import tpu_sc as plsc

EMBED_DIM = 64
LANES = 16
TILE_W = 128
CAP = 4096
SCAP = 512
SUPER = 35
FLUSH = 64


def _iota():
    return lax.iota(jnp.int32, LANES)


def _splat(x):
    return jnp.full((LANES,), x, jnp.int32)


@functools.lru_cache(maxsize=None)
def _make_extract(batch, num_rows):
    info = plsc.get_sparse_core_info()
    num_cores, num_subcores = info.num_cores, info.num_subcores
    num_workers = num_cores * num_subcores
    n_tiles = (num_rows + TILE_W - 1) // TILE_W
    tiles_per_w = (n_tiles + num_workers - 1) // num_workers
    mesh = plsc.VectorSubcoreMesh(core_axis_name="c", subcore_axis_name="s")
    rows_words = num_workers * CAP * EMBED_DIM

    @functools.partial(
        pl.kernel,
        mesh=mesh,
        out_type=[
            jax.ShapeDtypeStruct((rows_words,), jnp.float32),
            jax.ShapeDtypeStruct((rows_words,), jnp.float32),
            jax.ShapeDtypeStruct((batch + 1,), jnp.int32),
            jax.ShapeDtypeStruct((batch + 1,), jnp.int32),
        ],
        compiler_params=pltpu.CompilerParams(
            use_tc_tiling_on_sc=True, needs_layout_passes=False),
        scratch_types=[
            pltpu.VMEM((batch,), jnp.int32),
            pltpu.VMEM((CAP,), jnp.int32),
            pltpu.VMEM((CAP,), jnp.int32),
            pltpu.VMEM((SCAP,), jnp.int32),
            pltpu.VMEM((SCAP,), jnp.int32),
            pltpu.VMEM((2, 2, EMBED_DIM, TILE_W), jnp.float32),
            pltpu.VMEM((2 * FLUSH * EMBED_DIM,), jnp.float32),
            pltpu.VMEM((2 * FLUSH * EMBED_DIM,), jnp.float32),
            pltpu.VMEM((2 * CAP,), jnp.int32),
            pltpu.VMEM((LANES,), jnp.int32),
            pltpu.VMEM((2 * (CAP // LANES) * LANES,), jnp.int32),
            pltpu.SemaphoreType.DMA,
            pltpu.SemaphoreType.DMA,
            pltpu.SemaphoreType.DMA,
            pltpu.SemaphoreType.DMA,
            pltpu.SemaphoreType.DMA,
            pltpu.SemaphoreType.DMA,
            pltpu.SemaphoreType.DMA,
        ],
    )
    def extract(uid_hbm, iid_hbm, utT_hbm, itT_hbm,
                rows_u_hbm, rows_i_hbm, pmap_u_hbm, pmap_i_hbm,
                ids_v, hits_u, hits_i, slist_u, slist_i, stage_v,
                rowbuf_u, rowbuf_i, poslog_v, tmp_v, posval_v,
                sst0, sst1, sfu0, sfu1, sfi0, sfi1, spm):
        wid = lax.axis_index("s") * num_cores + lax.axis_index("c")
        t0 = wid * tiles_per_w
        ntiles = jnp.minimum(tiles_per_w, n_tiles - t0)
        ntiles = jnp.maximum(ntiles, 0)
        lo = t0 * TILE_W
        nr = ntiles * TILE_W
        lane0 = _iota() == 0
        wbase = wid * (CAP * EMBED_DIM)

        def pre(j, c):
            poslog_v[pl.ds(j * LANES, LANES)] = _splat(batch)
            return c
        lax.fori_loop(0, (2 * CAP) // LANES, pre, 0)

        def scan(ids_hbm, hits_ref):
            pltpu.sync_copy(ids_hbm, ids_v)

            def body(v, off):
                idv = ids_v[pl.ds(v * LANES, LANES)]
                rlo = idv - lo
                m = (rlo >= 0) & (rlo < nr)
                pk = lax.shift_left(rlo, 14) | (v * LANES + _iota())
                c16 = plsc.all_reduce_population_count(m)[0]
                can = off <= (CAP - LANES)

                @pl.when(can & (c16 > 0))
                def _st():
                    plsc.store_compressed(hits_ref.at[pl.ds(off, LANES)], pk, mask=m)
                return jnp.where(can, off + c16, off)

            return lax.fori_loop(0, batch // LANES, body, 0)

        n_u = scan(uid_hbm, hits_u)
        n_i = scan(iid_hbm, hits_i)

        stage_sems = (sst0, sst1)
        cvecs = [cb * LANES + _iota() for cb in range(EMBED_DIM // LANES)]

        def fire_tile(g, slot):
            off = pl.multiple_of(g * TILE_W, TILE_W)
            pltpu.async_copy(utT_hbm.at[:, pl.ds(off, TILE_W)],
                             stage_v.at[slot, 0], stage_sems[slot])
            pltpu.async_copy(itT_hbm.at[:, pl.ds(off, TILE_W)],
                             stage_v.at[slot, 1], stage_sems[slot])

        def drain_tile(slot):
            for tb in range(2):
                pltpu.make_async_copy(
                    utT_hbm.at[:, pl.ds(0, TILE_W)], stage_v.at[slot, tb],
                    stage_sems[slot]).wait()

        def flush(tbl, rowbuf, rows_hbm, f):
            sems = (sfu0, sfu1) if tbl == 0 else (sfi0, sfi1)
            par = f & 1
            for p in range(2):
                @pl.when(par == p)
                def _go(p=p):
                    @pl.when(f >= 2)
                    def _drain():
                        pltpu.make_async_copy(
                            rows_hbm.at[pl.ds(0, FLUSH * EMBED_DIM)],
                            rowbuf.at[pl.ds(p * FLUSH * EMBED_DIM,
                                            FLUSH * EMBED_DIM)],
                            sems[p]).wait()
                    pltpu.async_copy(
                        rowbuf.at[pl.ds(p * FLUSH * EMBED_DIM,
                                        FLUSH * EMBED_DIM)],
                        rows_hbm.at[pl.ds(wbase + f * (FLUSH * EMBED_DIM),
                                          FLUSH * EMBED_DIM)],
                        sems[p])

        def match_pass(tbl, slist, slen, blk_base, par, live, cnt0,
                       rowbuf, rows_hbm):
            def vec_body(v, cnt):
                hv = slist[pl.ds(v * LANES, LANES)]
                rhi = lax.shift_right_logical(hv, 14)
                valid = (v * LANES + _iota()) < slen
                m = valid & (rhi >= blk_base) & (rhi < blk_base + TILE_W) & live
                c16 = plsc.all_reduce_population_count(m)[0]

                @pl.when(c16 > 0)
                def _cp():
                    plsc.store_compressed(tmp_v.at[pl.ds(0, LANES)], hv, mask=m)

                def mb(j, cnt):
                    hj = plsc.load_gather(tmp_v, [_splat(j)])
                    lane = lax.shift_right_logical(hj, 14) & (TILE_W - 1)
                    b = hj & (2 ** 14 - 1)
                    slot = lax.shift_right_logical(cnt, 6) & 1
                    rbase = slot * (FLUSH * EMBED_DIM) + \
                        (cnt & (FLUSH - 1)) * EMBED_DIM
                    for cb in range(EMBED_DIM // LANES):
                        comp = plsc.load_gather(
                            stage_v.at[par, tbl], [cvecs[cb], lane])
                        rowbuf[pl.ds(rbase + cb * LANES, LANES)] = comp
                    plsc.store_scatter(
                        poslog_v, [_splat(tbl * CAP + cnt)], b, mask=lane0)
                    cnt = cnt + 1

                    @pl.when((cnt & (FLUSH - 1)) == 0)
                    def _fl():
                        flush(tbl, rowbuf, rows_hbm,
                              lax.shift_right_logical(cnt, 6) - 1)
                    return cnt

                return lax.fori_loop(0, c16, mb, cnt)

            return lax.fori_loop(0, (slen + LANES - 1) // LANES,
                                 vec_body, cnt0)

        n_supers = (ntiles + SUPER - 1) // SUPER

        def super_body(s, carry):
            cnt_u, cnt_i = carry
            stiles = jnp.minimum(SUPER, ntiles - s * SUPER)
            sbase = s * SUPER * TILE_W
            swidth = stiles * TILE_W

            def build(hits_ref, n_hits, slist):
                def body(v, off):
                    hv = hits_ref[pl.ds(v * LANES, LANES)]
                    rhi = lax.shift_right_logical(hv, 14)
                    valid = (v * LANES + _iota()) < n_hits
                    m = valid & (rhi >= sbase) & (rhi < sbase + swidth)
                    c16 = plsc.all_reduce_population_count(m)[0]
                    can = off <= (SCAP - LANES)

                    @pl.when(can & (c16 > 0))
                    def _st():
                        plsc.store_compressed(
                            slist.at[pl.ds(off, LANES)], hv, mask=m)
                    return jnp.where(can, off + c16, off)

                return lax.fori_loop(0, (n_hits + LANES - 1) // LANES, body, 0)

            slen_u = build(hits_u, n_u, slist_u)
            slen_i = build(hits_i, n_i, slist_i)

            @pl.when(stiles > 0)
            def _prime():
                fire_tile(t0 + s * SUPER, 0)

            def pair_body(p, carry):
                cnt_u, cnt_i = carry
                for par in range(2):
                    lt = p * 2 + par
                    live = lt < stiles

                    @pl.when(lt + 1 < stiles)
                    def _fire(lt=lt, par=par):
                        fire_tile(t0 + s * SUPER + lt + 1, 1 - par)

                    @pl.when(live)
                    def _drain(par=par):
                        drain_tile(par)
                    blk_base = sbase + lt * TILE_W
                    cnt_u = cnt_u + 0 * blk_base
                    cnt_i = cnt_i
                return cnt_u, cnt_i

            return lax.fori_loop(0, (stiles + 1) // 2, pair_body,
                                 (cnt_u, cnt_i))

        cnt_u, cnt_i = lax.fori_loop(0, n_supers, super_body, (0, 0))

        def finish(tbl, cnt, rowbuf, rows_hbm):
            sems = (sfu0, sfu1) if tbl == 0 else (sfi0, sfi1)

            @pl.when((cnt & (FLUSH - 1)) != 0)
            def _partial():
                flush(tbl, rowbuf, rows_hbm, lax.shift_right_logical(cnt, 6))
            total = lax.shift_right_logical(cnt + FLUSH - 1, 6)
            for k, guard in ((1, total >= 1), (2, total >= 2)):
                @pl.when(guard)
                def _dr(k=k):
                    par = (total - k) & 1
                    for p in range(2):
                        @pl.when(par == p)
                        def _w(p=p):
                            pltpu.make_async_copy(
                                rows_hbm.at[pl.ds(0, FLUSH * EMBED_DIM)],
                                rowbuf.at[pl.ds(p * FLUSH * EMBED_DIM,
                                                FLUSH * EMBED_DIM)],
                                sems[p]).wait()

        finish(0, cnt_u, rowbuf_u, rows_u_hbm)
        finish(1, cnt_i, rowbuf_i, rows_i_hbm)

        nch = CAP // LANES

        def pm_loop(tbl, poff, dst):
            def body(k, c):
                row = (tbl * nch + k) * LANES
                posval_v[pl.ds(row, LANES)] = wid * CAP + k * LANES + _iota()
                idx = poslog_v[pl.ds(poff + k * LANES, LANES)]
                pltpu.async_copy(posval_v.at[pl.ds(row, LANES)],
                                 dst.at[idx], spm)
                return c
            lax.fori_loop(0, nch, body, 0)

        pm_loop(0, 0, pmap_u_hbm)
        pm_loop(1, CAP, pmap_i_hbm)

        def pm_drain(k, c):
            pltpu.make_async_copy(
                pmap_u_hbm.at[pl.ds(0, LANES)],
                posval_v.at[pl.ds(0, LANES)], spm).wait()
            return c
        lax.fori_loop(0, 2 * nch, pm_drain, 0)

    return extract


@functools.lru_cache(maxsize=None)
def _make_dot(batch):
    info = plsc.get_sparse_core_info()
    num_cores, num_subcores = info.num_cores, info.num_subcores
    num_workers = num_cores * num_subcores
    b_per_w = batch // num_workers
    mesh = plsc.VectorSubcoreMesh(core_axis_name="c", subcore_axis_name="s")

    @functools.partial(
        pl.kernel,
        mesh=mesh,
        out_type=jax.ShapeDtypeStruct((batch,), jnp.float32),
        compiler_params=pltpu.CompilerParams(
            use_tc_tiling_on_sc=False, needs_layout_passes=False),
        scratch_types=[
            pltpu.VMEM((b_per_w,), jnp.int32),
            pltpu.VMEM((b_per_w,), jnp.int32),
            pltpu.VMEM((b_per_w, EMBED_DIM), jnp.float32),
            pltpu.VMEM((b_per_w, EMBED_DIM), jnp.float32),
            pltpu.VMEM((b_per_w,), jnp.float32),
            pltpu.SemaphoreType.DMA,
        ],
    )
    def dot_k(pmu_hbm, pmi_hbm, ru_hbm, ri_hbm, out_hbm,
              pmu_v, pmi_v, rows_u_v, rows_i_v, scores_v, sem):
        wid = lax.axis_index("s") * num_cores + lax.axis_index("c")
        base = wid * b_per_w
        pltpu.sync_copy(pmu_hbm.at[pl.ds(base, b_per_w)], pmu_v)
        pltpu.sync_copy(pmi_hbm.at[pl.ds(base, b_per_w)], pmi_v)

        copies = []
        for k in range(b_per_w // LANES):
            s = pl.ds(k * LANES, LANES)
            copies.append(pltpu.async_copy(
                ru_hbm.at[pmu_v[s]], rows_u_v.at[s], sem))
            copies.append(pltpu.async_copy(
                ri_hbm.at[pmi_v[s]], rows_i_v.at[s], sem))
        for cp in copies:
            cp.wait()

        lane = _iota()

        def group(g, carry):
            rows = g * LANES + lane
            acc = jnp.zeros((LANES,), jnp.float32)
            for d in range(EMBED_DIM):
                col = _splat(d)
                u = plsc.load_gather(rows_u_v, [rows, col])
                v = plsc.load_gather(rows_i_v, [rows, col])
                acc = acc + u * v
            scores_v[pl.ds(g * LANES, LANES)] = acc
            return carry

        lax.fori_loop(0, b_per_w // LANES, group, 0)
        pltpu.sync_copy(scores_v, out_hbm.at[pl.ds(base, b_per_w)])

    return dot_k


def kernel(user_ids, item_ids, user_table, item_table):
    batch = user_ids.shape[0]
    num_rows = user_table.shape[0]
    k1 = _make_extract(batch, num_rows)
    rows_u, rows_i, pmap_u, pmap_i = k1(
        user_ids, item_ids, user_table.T, item_table.T)
    ru2 = rows_u.reshape(-1, EMBED_DIM)
    ri2 = rows_i.reshape(-1, EMBED_DIM)
    k2 = _make_dot(batch)
    return k2(pmap_u, pmap_i, ru2, ri2)


# speedup vs baseline: 94.4015x; 94.4015x over previous
import functools

import jax
import jax.numpy as jnp
from jax import lax
from jax.experimental import pallas as pl
from jax.experimental.pallas import tpu as pltpu
from jax.experimental.pallas import tpu_sc as plsc

EMBED_DIM = 64
LANES = 16
TILE_W = 128
CHUNK = 1
SLOTS = 6
PREF = 4


@functools.lru_cache(maxsize=None)
def _make_kernel(batch, num_rows):
    info = plsc.get_sparse_core_info()
    num_cores, num_subcores = info.num_cores, info.num_subcores
    num_workers = num_cores * num_subcores
    b_per_w = batch // num_workers
    n_chunks = b_per_w // CHUNK
    assert b_per_w % CHUNK == 0
    mesh = plsc.VectorSubcoreMesh(core_axis_name="c", subcore_axis_name="s")

    lane_iota = lambda: lax.iota(jnp.int32, LANES)

    @functools.partial(
        pl.kernel,
        mesh=mesh,
        out_type=jax.ShapeDtypeStruct((batch,), jnp.float32),
        compiler_params=pltpu.CompilerParams(
            use_tc_tiling_on_sc=True, needs_layout_passes=False),
        scratch_types=[
            pltpu.VMEM((b_per_w,), jnp.int32),
            pltpu.VMEM((b_per_w,), jnp.int32),
            pltpu.VMEM((b_per_w,), jnp.int32),
            pltpu.VMEM((b_per_w,), jnp.int32),
            pltpu.VMEM((SLOTS, CHUNK, EMBED_DIM, TILE_W), jnp.float32),
            pltpu.VMEM((SLOTS, CHUNK, EMBED_DIM, TILE_W), jnp.float32),
            pltpu.VMEM((b_per_w,), jnp.float32),
        ] + [pltpu.SemaphoreType.DMA] * SLOTS,
    )
    def mf_bpr(uid_hbm, iid_hbm, utT_hbm, itT_hbm, out_hbm,
               utoff_v, itoff_v, ulane_v, ilane_v, ubuf_v, ibuf_v,
               scores_v, *sems):
        wid = lax.axis_index("s") * num_cores + lax.axis_index("c")
        base = wid * b_per_w
        pltpu.sync_copy(uid_hbm.at[pl.ds(base, b_per_w)], utoff_v)
        pltpu.sync_copy(iid_hbm.at[pl.ds(base, b_per_w)], itoff_v)

        def split(v, _):
            s = pl.ds(v * LANES, LANES)
            ru = utoff_v[s]
            ri = itoff_v[s]
            ulane_v[s] = ru & (TILE_W - 1)
            ilane_v[s] = ri & (TILE_W - 1)
            utoff_v[s] = ru - (ru & (TILE_W - 1))
            itoff_v[s] = ri - (ri & (TILE_W - 1))
            return _

        lax.fori_loop(0, b_per_w // LANES, split, 0)


        def splat_at(ref, i):
            return plsc.load_gather(ref, [jnp.full((LANES,), i, jnp.int32)])

        def fire(c, slot):
            sem = sems[slot]
            for j in range(CHUNK):
                i = c * CHUNK + j
                tu = splat_at(utoff_v, i)[0]
                ti = splat_at(itoff_v, i)[0]
                pltpu.async_copy(
                    utT_hbm.at[:, pl.ds(pl.multiple_of(tu, TILE_W), TILE_W)],
                    ubuf_v.at[slot, j], sem)
                pltpu.async_copy(
                    itT_hbm.at[:, pl.ds(pl.multiple_of(ti, TILE_W), TILE_W)],
                    ibuf_v.at[slot, j], sem)

        def drain(slot):
            sem = sems[slot]
            for j in range(CHUNK):
                pltpu.make_async_copy(
                    utT_hbm.at[:, pl.ds(0, TILE_W)], ubuf_v.at[slot, j], sem
                ).wait()
                pltpu.make_async_copy(
                    itT_hbm.at[:, pl.ds(0, TILE_W)], ibuf_v.at[slot, j], sem
                ).wait()

        def extract(c, slot):
            lane0 = lane_iota() == 0
            for j in range(CHUNK):
                i = c * CHUNK + j
                ul = splat_at(ulane_v, i)
                il = splat_at(ilane_v, i)
                acc = None
                for cb in range(EMBED_DIM // LANES):
                    cvec = cb * LANES + lane_iota()
                    u = plsc.load_gather(ubuf_v.at[slot, j], [cvec, ul])
                    v = plsc.load_gather(ibuf_v.at[slot, j], [cvec, il])
                    acc = u * v if acc is None else acc + u * v
                score = jnp.full((LANES,), jnp.sum(acc), jnp.float32)
                plsc.store_scatter(
                    scores_v, [jnp.full((LANES,), i, jnp.int32)], score,
                    mask=lane0)

        for k in range(PREF):
            fire(k, k)

        def step(c, fire_slot, dx_slot):
            @pl.when(c + PREF < n_chunks)
            def _fire_next():
                fire(c + PREF, fire_slot)

            @pl.when(c < n_chunks)
            def _dx():
                drain(dx_slot)
                extract(c, dx_slot)

        def body(p, _):
            c = p * SLOTS
            for q in range(SLOTS):
                step(c + q, (q + PREF) % SLOTS, q)
            return _

        lax.fori_loop(0, (n_chunks + SLOTS - 1) // SLOTS, body, 0)
        pltpu.sync_copy(scores_v, out_hbm.at[pl.ds(base, b_per_w)])

    return mf_bpr


def kernel(user_ids, item_ids, user_table, item_table):
    batch = user_ids.shape[0]
    k = _make_kernel(batch, user_table.shape[0])
    return k(user_ids, item_ids, user_table.T, item_table.T)


# speedup vs baseline: 99.2063x; 1.0509x over previous
import functools

import jax
import jax.numpy as jnp
from jax import lax
from jax.experimental import pallas as pl
from jax.experimental.pallas import tpu as pltpu
from jax.experimental.pallas import tpu_sc as plsc

EMBED_DIM = 64
LANES = 16
TILE_W = 128
CHUNK = 1
SLOTS = 7
PREF = 5


@functools.lru_cache(maxsize=None)
def _make_kernel(batch, num_rows):
    info = plsc.get_sparse_core_info()
    num_cores, num_subcores = info.num_cores, info.num_subcores
    num_workers = num_cores * num_subcores
    b_per_w = batch // num_workers
    n_chunks = b_per_w // CHUNK
    assert b_per_w % CHUNK == 0
    mesh = plsc.VectorSubcoreMesh(core_axis_name="c", subcore_axis_name="s")

    lane_iota = lambda: lax.iota(jnp.int32, LANES)

    @functools.partial(
        pl.kernel,
        mesh=mesh,
        out_type=jax.ShapeDtypeStruct((batch,), jnp.float32),
        compiler_params=pltpu.CompilerParams(
            use_tc_tiling_on_sc=True, needs_layout_passes=False),
        scratch_types=[
            pltpu.VMEM((b_per_w,), jnp.int32),
            pltpu.VMEM((b_per_w,), jnp.int32),
            pltpu.VMEM((b_per_w,), jnp.int32),
            pltpu.VMEM((b_per_w,), jnp.int32),
            pltpu.VMEM((SLOTS, CHUNK, EMBED_DIM, TILE_W), jnp.float32),
            pltpu.VMEM((SLOTS, CHUNK, EMBED_DIM, TILE_W), jnp.float32),
            pltpu.VMEM((b_per_w,), jnp.float32),
        ] + [pltpu.SemaphoreType.DMA] * SLOTS,
    )
    def mf_bpr(uid_hbm, iid_hbm, utT_hbm, itT_hbm, out_hbm,
               utoff_v, itoff_v, ulane_v, ilane_v, ubuf_v, ibuf_v,
               scores_v, *sems):
        wid = lax.axis_index("s") * num_cores + lax.axis_index("c")
        base = wid * b_per_w
        pltpu.sync_copy(uid_hbm.at[pl.ds(base, b_per_w)], utoff_v)
        pltpu.sync_copy(iid_hbm.at[pl.ds(base, b_per_w)], itoff_v)

        def split(v, _):
            s = pl.ds(v * LANES, LANES)
            ru = utoff_v[s]
            ri = itoff_v[s]
            ulane_v[s] = ru & (TILE_W - 1)
            ilane_v[s] = ri & (TILE_W - 1)
            utoff_v[s] = ru - (ru & (TILE_W - 1))
            itoff_v[s] = ri - (ri & (TILE_W - 1))
            return _

        lax.fori_loop(0, b_per_w // LANES, split, 0)


        def splat_at(ref, i):
            return plsc.load_gather(ref, [jnp.full((LANES,), i, jnp.int32)])

        def fire(c, slot):
            sem = sems[slot]
            for j in range(CHUNK):
                i = c * CHUNK + j
                tu = splat_at(utoff_v, i)[0]
                ti = splat_at(itoff_v, i)[0]
                pltpu.async_copy(
                    utT_hbm.at[:, pl.ds(pl.multiple_of(tu, TILE_W), TILE_W)],
                    ubuf_v.at[slot, j], sem)
                pltpu.async_copy(
                    itT_hbm.at[:, pl.ds(pl.multiple_of(ti, TILE_W), TILE_W)],
                    ibuf_v.at[slot, j], sem)

        def drain(slot):
            sem = sems[slot]
            for j in range(CHUNK):
                pltpu.make_async_copy(
                    utT_hbm.at[:, pl.ds(0, TILE_W)], ubuf_v.at[slot, j], sem
                ).wait()
                pltpu.make_async_copy(
                    itT_hbm.at[:, pl.ds(0, TILE_W)], ibuf_v.at[slot, j], sem
                ).wait()

        def extract(c, slot):
            lane0 = lane_iota() == 0
            for j in range(CHUNK):
                i = c * CHUNK + j
                ul = splat_at(ulane_v, i)
                il = splat_at(ilane_v, i)
                acc = None
                for cb in range(EMBED_DIM // LANES):
                    cvec = cb * LANES + lane_iota()
                    u = plsc.load_gather(ubuf_v.at[slot, j], [cvec, ul])
                    v = plsc.load_gather(ibuf_v.at[slot, j], [cvec, il])
                    acc = u * v if acc is None else acc + u * v
                score = jnp.full((LANES,), jnp.sum(acc), jnp.float32)
                plsc.store_scatter(
                    scores_v, [jnp.full((LANES,), i, jnp.int32)], score,
                    mask=lane0)

        for k in range(PREF):
            fire(k, k)

        def step(c, fire_slot, dx_slot):
            @pl.when(c + PREF < n_chunks)
            def _fire_next():
                fire(c + PREF, fire_slot)

            @pl.when(c < n_chunks)
            def _dx():
                drain(dx_slot)
                extract(c, dx_slot)

        def body(p, _):
            c = p * SLOTS
            for q in range(SLOTS):
                step(c + q, (q + PREF) % SLOTS, q)
            return _

        lax.fori_loop(0, (n_chunks + SLOTS - 1) // SLOTS, body, 0)
        pltpu.sync_copy(scores_v, out_hbm.at[pl.ds(base, b_per_w)])

    return mf_bpr


def kernel(user_ids, item_ids, user_table, item_table):
    batch = user_ids.shape[0]
    k = _make_kernel(batch, user_table.shape[0])
    return k(user_ids, item_ids, user_table.T, item_table.T)
